# baseline (speedup 1.0000x reference)
import functools

import jax
import jax.numpy as jnp
from jax import lax
from jax.experimental import pallas as pl
from jax.experimental.pallas import tpu as pltpu
from jax.experimental.pallas import tpu_sc as plsc

N = 10000
E = 320000
D = 128

NC = 2
NS = 16
NW = NC * NS
EPW = E // NW
CH = 80
NCH = EPW // CH
NPAD = 10112
RPT = NPAD // NS

_mesh = plsc.VectorSubcoreMesh(core_axis_name="c", subcore_axis_name="s")


@functools.partial(
    pl.kernel,
    out_type=jax.ShapeDtypeStruct((NW, NPAD), jnp.float32),
    mesh=_mesh,
    scratch_types=[
        pltpu.VMEM((EPW,), jnp.int32),
        pltpu.VMEM((NPAD,), jnp.float32),
    ],
    compiler_params=pltpu.CompilerParams(needs_layout_passes=False),
)
def _deg_kernel(dst_hbm, zeros_hbm, out_hbm, dst_v, deg_v):
    c = lax.axis_index("c")
    s = lax.axis_index("s")
    wid = s * NC + c
    pltpu.sync_copy(dst_hbm.at[pl.ds(wid * EPW, EPW)], dst_v)
    pltpu.sync_copy(zeros_hbm, deg_v)
    ones = jnp.full((16,), 1.0, dtype=jnp.float32)

    def body(i, carry):
        base = i * 80
        for u in range(5):
            idx = dst_v[pl.ds(base + u * 16, 16)]
            plsc.addupdate_scatter(deg_v, [idx], ones)
        return carry

    lax.fori_loop(0, EPW // 80, body, 0)
    pltpu.sync_copy(deg_v, out_hbm.at[wid])


@functools.partial(
    pl.kernel,
    out_type=jax.ShapeDtypeStruct((NC, NPAD, D), jnp.float32),
    mesh=_mesh,
    scratch_types=[
        pltpu.VMEM((EPW,), jnp.int32),
        pltpu.VMEM((NCH, CH), jnp.int32),
        pltpu.VMEM((CH, D), jnp.float32),
        pltpu.VMEM((CH, D), jnp.float32),
        pltpu.SemaphoreType.DMA,
        pltpu.SemaphoreType.DMA,
        pltpu.VMEM_SHARED((NPAD, D), jnp.float32),
    ],
)
def _agg_kernel(h_hbm, src_hbm, dst_hbm, zrows_hbm, out_hbm,
                src_v, dst_v, rows_a, rows_b, sem_a, sem_b, acc_sh):
    c = lax.axis_index("c")
    s = lax.axis_index("s")
    wid = s * NC + c
    pltpu.sync_copy(zrows_hbm, acc_sh.at[pl.ds(s * RPT, RPT)])
    pltpu.sync_copy(src_hbm.at[wid], src_v)
    pltpu.sync_copy(dst_hbm.at[wid], dst_v)
    plsc.subcore_barrier()

    def g(j, rows, sem):
        return pltpu.make_async_copy(
            h_hbm.at[src_v.at[pl.ds(j * CH, CH)]], rows, sem)

    g(0, rows_a, sem_a).start()

    def step(k, carry):
        j0 = 2 * k
        g(j0 + 1, rows_b, sem_b).start()
        g(j0, rows_a, sem_a).wait()
        pltpu.sync_copy(rows_a, acc_sh.at[dst_v.at[j0]], add=True)
        g(j0 + 2, rows_a, sem_a).start()
        g(j0 + 1, rows_b, sem_b).wait()
        pltpu.sync_copy(rows_b, acc_sh.at[dst_v.at[j0 + 1]], add=True)
        return carry

    lax.fori_loop(0, (NCH - 1) // 2, step, 0)
    g(NCH - 1, rows_a, sem_a).wait()
    pltpu.sync_copy(rows_a, acc_sh.at[dst_v.at[NCH - 1]], add=True)
    plsc.subcore_barrier()
    pltpu.sync_copy(acc_sh.at[pl.ds(s * RPT, RPT)],
                    out_hbm.at[c, pl.ds(s * RPT, RPT)])


def _tc0_body(x_ref, w1_ref, h1_ref):
    h1_ref[...] = jnp.dot(x_ref[...], w1_ref[...],
                          preferred_element_type=jnp.float32)


def _tc0(x, w1):
    return pl.pallas_call(
        _tc0_body,
        out_shape=jax.ShapeDtypeStruct((N, D), jnp.float32),
    )(x, w1)


def _tc1_body(h1_ref, degp_ref, h1p_ref, dis_ref):
    degp = degp_ref[...]
    ones = jnp.ones((NW, 1), dtype=jnp.float32)
    deg = lax.dot_general(degp, ones, (((0,), (0,)), ((), ())),
                          preferred_element_type=jnp.float32) + 1.0
    dis = lax.rsqrt(deg)
    h1p_ref[...] = h1_ref[...] * dis[:N]
    dis_ref[...] = dis


def _tc1(h1, degp):
    return pl.pallas_call(
        _tc1_body,
        out_shape=(
            jax.ShapeDtypeStruct((N, D), jnp.float32),
            jax.ShapeDtypeStruct((NPAD, 1), jnp.float32),
        ),
    )(h1, degp)


def _tc2_body(acc_ref, h1p_ref, dis_ref, w2_ref, b1_ref, g_ref, be_ref,
              h2p_ref):
    dis = dis_ref[...][:N]
    acc = acc_ref[0] + acc_ref[1]
    out1 = (acc[:N] + h1p_ref[...]) * dis + b1_ref[...]
    mean = jnp.mean(out1, axis=0, keepdims=True)
    var = jnp.mean((out1 - mean) * (out1 - mean), axis=0, keepdims=True)
    hbn = (out1 - mean) * lax.rsqrt(var + 1e-5) * g_ref[...] + be_ref[...]
    hr = jnp.maximum(hbn, 0.0)
    h2 = jnp.dot(hr, w2_ref[...], preferred_element_type=jnp.float32)
    h2p_ref[...] = h2 * dis


def _tc2(acc, h1p, dis, w2, b1, g, be):
    return pl.pallas_call(
        _tc2_body,
        out_shape=jax.ShapeDtypeStruct((N, D), jnp.float32),
    )(acc, h1p, dis, w2, b1, g, be)


def _tc3_body(acc_ref, h2p_ref, dis_ref, b2_ref, out_ref):
    dis = dis_ref[...][:N]
    acc = acc_ref[0] + acc_ref[1]
    out_ref[...] = (acc[:N] + h2p_ref[...]) * dis + b2_ref[...]


def _tc3(acc, h2p, dis, b2):
    return pl.pallas_call(
        _tc3_body,
        out_shape=jax.ShapeDtypeStruct((N, D), jnp.float32),
    )(acc, h2p, dis, b2)


@jax.jit
def kernel(x, edge_index, W1, b1, gamma1, beta1, W2, b2):
    src = edge_index[0].reshape(NW, EPW)
    dst = edge_index[1].reshape(NW, NCH, CH)
    zeros_n = jnp.zeros((NPAD,), dtype=jnp.float32)
    zeros_rows = jnp.zeros((RPT, D), dtype=jnp.float32)

    degp = _deg_kernel(edge_index[1], zeros_n)
    h1 = _tc0(x, W1)
    h1p, dis = _tc1(h1, degp)
    acc1 = _agg_kernel(h1p, src, dst, zeros_rows)
    h2p = _tc2(acc1, h1p, dis, W2, b1.reshape(1, D), gamma1.reshape(1, D),
               beta1.reshape(1, D))
    acc2 = _agg_kernel(h2p, src, dst, zeros_rows)
    return _tc3(acc2, h2p, dis, b2.reshape(1, D))


# speedup vs baseline: 1.0571x; 1.0571x over previous
import functools

import jax
import jax.numpy as jnp
from jax import lax
from jax.experimental import pallas as pl
from jax.experimental.pallas import tpu as pltpu
from jax.experimental.pallas import tpu_sc as plsc

N = 10000
E = 320000
D = 128

NC = 2
NS = 16
NW = NC * NS
EPW = E // NW
CH = 104
NCH = 96
TAIL = EPW - NCH * CH
NPAD = 10112
RPT = NPAD // NS

_mesh = plsc.VectorSubcoreMesh(core_axis_name="c", subcore_axis_name="s")


@functools.partial(
    pl.kernel,
    out_type=jax.ShapeDtypeStruct((NW, NPAD), jnp.float32),
    mesh=_mesh,
    scratch_types=[
        pltpu.VMEM((EPW,), jnp.int32),
        pltpu.VMEM((NPAD,), jnp.float32),
    ],
    compiler_params=pltpu.CompilerParams(needs_layout_passes=False),
)
def _deg_kernel(dst_hbm, zeros_hbm, out_hbm, dst_v, deg_v):
    c = lax.axis_index("c")
    s = lax.axis_index("s")
    wid = s * NC + c
    pltpu.sync_copy(dst_hbm.at[pl.ds(wid * EPW, EPW)], dst_v)
    pltpu.sync_copy(zeros_hbm, deg_v)
    ones = jnp.full((16,), 1.0, dtype=jnp.float32)

    def body(i, carry):
        base = i * 80
        for u in range(5):
            idx = dst_v[pl.ds(base + u * 16, 16)]
            plsc.addupdate_scatter(deg_v, [idx], ones)
        return carry

    lax.fori_loop(0, EPW // 80, body, 0)
    pltpu.sync_copy(deg_v, out_hbm.at[wid])


@functools.partial(
    pl.kernel,
    out_type=jax.ShapeDtypeStruct((NC, NPAD, D), jnp.float32),
    mesh=_mesh,
    scratch_types=[
        pltpu.VMEM((EPW,), jnp.int32),
        pltpu.VMEM((NCH, CH), jnp.int32),
        pltpu.VMEM((TAIL,), jnp.int32),
        pltpu.VMEM((CH, D), jnp.float32),
        pltpu.VMEM((CH, D), jnp.float32),
        pltpu.SemaphoreType.DMA,
        pltpu.SemaphoreType.DMA,
        pltpu.VMEM_SHARED((NPAD, D), jnp.float32),
    ],
)
def _agg_kernel(h_hbm, src_hbm, dst_hbm, dstt_hbm, zrows_hbm, out_hbm,
                src_v, dst_v, dstt_v, rows_a, rows_b, sem_a, sem_b, acc_sh):
    c = lax.axis_index("c")
    s = lax.axis_index("s")
    wid = s * NC + c
    pltpu.sync_copy(zrows_hbm, acc_sh.at[pl.ds(s * RPT, RPT)])
    pltpu.sync_copy(src_hbm.at[wid], src_v)
    pltpu.sync_copy(dst_hbm.at[wid], dst_v)
    pltpu.sync_copy(dstt_hbm.at[wid], dstt_v)
    plsc.subcore_barrier()

    def g(j, rows, sem):
        return pltpu.make_async_copy(
            h_hbm.at[src_v.at[pl.ds(j * CH, CH)]], rows, sem)

    g(0, rows_a, sem_a).start()

    def step(k, carry):
        j0 = 2 * k
        g(j0 + 1, rows_b, sem_b).start()
        g(j0, rows_a, sem_a).wait()
        pltpu.sync_copy(rows_a, acc_sh.at[dst_v.at[j0]], add=True)
        g(j0 + 2, rows_a, sem_a).start()
        g(j0 + 1, rows_b, sem_b).wait()
        pltpu.sync_copy(rows_b, acc_sh.at[dst_v.at[j0 + 1]], add=True)
        return carry

    lax.fori_loop(0, (NCH - 2) // 2, step, 0)
    g(NCH - 1, rows_b, sem_b).start()
    g(NCH - 2, rows_a, sem_a).wait()
    pltpu.sync_copy(rows_a, acc_sh.at[dst_v.at[NCH - 2]], add=True)
    g(NCH - 1, rows_b, sem_b).wait()
    pltpu.sync_copy(rows_b, acc_sh.at[dst_v.at[NCH - 1]], add=True)
    tg = pltpu.make_async_copy(
        h_hbm.at[src_v.at[pl.ds(NCH * CH, TAIL)]],
        rows_a.at[pl.ds(0, TAIL)], sem_a)
    tg.start()
    tg.wait()
    pltpu.sync_copy(rows_a.at[pl.ds(0, TAIL)], acc_sh.at[dstt_v], add=True)
    plsc.subcore_barrier()
    pltpu.sync_copy(acc_sh.at[pl.ds(s * RPT, RPT)],
                    out_hbm.at[c, pl.ds(s * RPT, RPT)])


def _tc1_body(x_ref, w1_ref, degp_ref, h1p_ref, dis_ref):
    degp = degp_ref[...]
    ones = jnp.ones((NW, 1), dtype=jnp.float32)
    deg = lax.dot_general(degp, ones, (((0,), (0,)), ((), ())),
                          preferred_element_type=jnp.float32) + 1.0
    dis = lax.rsqrt(deg)
    h1 = jnp.dot(x_ref[...], w1_ref[...],
                 preferred_element_type=jnp.float32)
    h1p_ref[...] = h1 * dis[:N]
    dis_ref[...] = dis


def _tc1(x, w1, degp):
    return pl.pallas_call(
        _tc1_body,
        out_shape=(
            jax.ShapeDtypeStruct((N, D), jnp.float32),
            jax.ShapeDtypeStruct((NPAD, 1), jnp.float32),
        ),
    )(x, w1, degp)


def _tc2_body(acc_ref, h1p_ref, dis_ref, w2_ref, b1_ref, g_ref, be_ref,
              h2p_ref):
    dis = dis_ref[...][:N]
    acc = acc_ref[0] + acc_ref[1]
    out1 = (acc[:N] + h1p_ref[...]) * dis + b1_ref[...]
    mean = jnp.mean(out1, axis=0, keepdims=True)
    var = jnp.mean((out1 - mean) * (out1 - mean), axis=0, keepdims=True)
    hbn = (out1 - mean) * lax.rsqrt(var + 1e-5) * g_ref[...] + be_ref[...]
    hr = jnp.maximum(hbn, 0.0)
    h2 = jnp.dot(hr, w2_ref[...], preferred_element_type=jnp.float32)
    h2p_ref[...] = h2 * dis


def _tc2(acc, h1p, dis, w2, b1, g, be):
    return pl.pallas_call(
        _tc2_body,
        out_shape=jax.ShapeDtypeStruct((N, D), jnp.float32),
    )(acc, h1p, dis, w2, b1, g, be)


def _tc3_body(acc_ref, h2p_ref, dis_ref, b2_ref, out_ref):
    dis = dis_ref[...][:N]
    acc = acc_ref[0] + acc_ref[1]
    out_ref[...] = (acc[:N] + h2p_ref[...]) * dis + b2_ref[...]


def _tc3(acc, h2p, dis, b2):
    return pl.pallas_call(
        _tc3_body,
        out_shape=jax.ShapeDtypeStruct((N, D), jnp.float32),
    )(acc, h2p, dis, b2)


@jax.jit
def kernel(x, edge_index, W1, b1, gamma1, beta1, W2, b2):
    src = edge_index[0].reshape(NW, EPW)
    dst_w = edge_index[1].reshape(NW, EPW)
    dst = dst_w[:, :NCH * CH].reshape(NW, NCH, CH)
    dstt = dst_w[:, NCH * CH:]
    zeros_n = jnp.zeros((NPAD,), dtype=jnp.float32)
    zeros_rows = jnp.zeros((RPT, D), dtype=jnp.float32)

    degp = _deg_kernel(edge_index[1], zeros_n)
    h1p, dis = _tc1(x, W1, degp)
    acc1 = _agg_kernel(h1p, src, dst, dstt, zeros_rows)
    h2p = _tc2(acc1, h1p, dis, W2, b1.reshape(1, D), gamma1.reshape(1, D),
               beta1.reshape(1, D))
    acc2 = _agg_kernel(h2p, src, dst, dstt, zeros_rows)
    return _tc3(acc2, h2p, dis, b2.reshape(1, D))


# speedup vs baseline: 1.0741x; 1.0160x over previous
import functools

import jax
import jax.numpy as jnp
from jax import lax
from jax.experimental import pallas as pl
from jax.experimental.pallas import tpu as pltpu
from jax.experimental.pallas import tpu_sc as plsc

N = 10000
E = 320000
D = 128

NC = 2
NS = 16
NW = NC * NS
EPW = E // NW
CH = 104
NCH = 96
TAIL = EPW - NCH * CH
NPAD = 10112
RPT = NPAD // NS

_mesh = plsc.VectorSubcoreMesh(core_axis_name="c", subcore_axis_name="s")


@functools.partial(
    pl.kernel,
    out_type=jax.ShapeDtypeStruct((NW, NPAD), jnp.float32),
    mesh=_mesh,
    scratch_types=[
        pltpu.VMEM((EPW,), jnp.int32),
        pltpu.VMEM((NPAD,), jnp.float32),
    ],
    compiler_params=pltpu.CompilerParams(needs_layout_passes=False),
)
def _deg_kernel(dst_hbm, zeros_hbm, out_hbm, dst_v, deg_v):
    c = lax.axis_index("c")
    s = lax.axis_index("s")
    wid = s * NC + c
    pltpu.sync_copy(dst_hbm.at[pl.ds(wid * EPW, EPW)], dst_v)
    pltpu.sync_copy(zeros_hbm, deg_v)
    ones = jnp.full((16,), 1.0, dtype=jnp.float32)

    def body(i, carry):
        base = i * 80
        for u in range(5):
            idx = dst_v[pl.ds(base + u * 16, 16)]
            plsc.addupdate_scatter(deg_v, [idx], ones)
        return carry

    lax.fori_loop(0, EPW // 80, body, 0)
    pltpu.sync_copy(deg_v, out_hbm.at[wid])


@functools.partial(
    pl.kernel,
    out_type=jax.ShapeDtypeStruct((NC, NPAD, D), jnp.float32),
    mesh=_mesh,
    scratch_types=[
        pltpu.VMEM((EPW,), jnp.int32),
        pltpu.VMEM((NCH, CH), jnp.int32),
        pltpu.VMEM((TAIL,), jnp.int32),
        pltpu.VMEM((CH, D), jnp.float32),
        pltpu.VMEM((CH, D), jnp.float32),
        pltpu.SemaphoreType.DMA,
        pltpu.SemaphoreType.DMA,
        pltpu.VMEM_SHARED((NPAD, D), jnp.float32),
    ],
)
def _agg_kernel(h_hbm, src_hbm, dst_hbm, dstt_hbm, zrows_hbm, out_hbm,
                src_v, dst_v, dstt_v, rows_a, rows_b, sem_a, sem_b, acc_sh):
    c = lax.axis_index("c")
    s = lax.axis_index("s")
    wid = s * NC + c
    p0 = pltpu.make_async_copy(zrows_hbm, acc_sh.at[pl.ds(s * RPT, RPT)],
                               sem_a)
    p1 = pltpu.make_async_copy(src_hbm.at[wid], src_v, sem_b)
    p2 = pltpu.make_async_copy(dst_hbm.at[wid], dst_v, sem_a)
    p3 = pltpu.make_async_copy(dstt_hbm.at[wid], dstt_v, sem_b)
    p0.start()
    p1.start()
    p2.start()
    p3.start()
    p0.wait()
    p1.wait()
    p2.wait()
    p3.wait()
    plsc.subcore_barrier()

    def g(j, rows, sem):
        return pltpu.make_async_copy(
            h_hbm.at[src_v.at[pl.ds(j * CH, CH)]], rows, sem)

    g(0, rows_a, sem_a).start()

    def step(k, carry):
        j0 = 2 * k
        g(j0 + 1, rows_b, sem_b).start()
        g(j0, rows_a, sem_a).wait()
        pltpu.sync_copy(rows_a, acc_sh.at[dst_v.at[j0]], add=True)
        g(j0 + 2, rows_a, sem_a).start()
        g(j0 + 1, rows_b, sem_b).wait()
        pltpu.sync_copy(rows_b, acc_sh.at[dst_v.at[j0 + 1]], add=True)
        return carry

    lax.fori_loop(0, (NCH - 2) // 2, step, 0)
    g(NCH - 1, rows_b, sem_b).start()
    g(NCH - 2, rows_a, sem_a).wait()
    pltpu.sync_copy(rows_a, acc_sh.at[dst_v.at[NCH - 2]], add=True)
    g(NCH - 1, rows_b, sem_b).wait()
    pltpu.sync_copy(rows_b, acc_sh.at[dst_v.at[NCH - 1]], add=True)
    tg = pltpu.make_async_copy(
        h_hbm.at[src_v.at[pl.ds(NCH * CH, TAIL)]],
        rows_a.at[pl.ds(0, TAIL)], sem_a)
    tg.start()
    tg.wait()
    pltpu.sync_copy(rows_a.at[pl.ds(0, TAIL)], acc_sh.at[dstt_v], add=True)
    plsc.subcore_barrier()
    pltpu.sync_copy(acc_sh.at[pl.ds(s * RPT, RPT)],
                    out_hbm.at[c, pl.ds(s * RPT, RPT)])


def _tc1_body(x_ref, w1_ref, degp_ref, h1p_ref, dis_ref):
    degp = degp_ref[...]
    ones = jnp.ones((NW, 1), dtype=jnp.float32)
    deg = lax.dot_general(degp, ones, (((0,), (0,)), ((), ())),
                          preferred_element_type=jnp.float32) + 1.0
    dis = lax.rsqrt(deg)
    h1 = jnp.dot(x_ref[...], w1_ref[...],
                 preferred_element_type=jnp.float32)
    h1p_ref[...] = h1 * dis[:N]
    dis_ref[...] = dis


def _tc1(x, w1, degp):
    return pl.pallas_call(
        _tc1_body,
        out_shape=(
            jax.ShapeDtypeStruct((N, D), jnp.float32),
            jax.ShapeDtypeStruct((NPAD, 1), jnp.float32),
        ),
    )(x, w1, degp)


def _tc2_body(acc_ref, h1p_ref, dis_ref, w2_ref, b1_ref, g_ref, be_ref,
              h2p_ref):
    dis = dis_ref[...][:N]
    acc = acc_ref[0] + acc_ref[1]
    out1 = (acc[:N] + h1p_ref[...]) * dis + b1_ref[...]
    mean = jnp.mean(out1, axis=0, keepdims=True)
    var = jnp.mean((out1 - mean) * (out1 - mean), axis=0, keepdims=True)
    hbn = (out1 - mean) * lax.rsqrt(var + 1e-5) * g_ref[...] + be_ref[...]
    hr = jnp.maximum(hbn, 0.0)
    h2 = jnp.dot(hr, w2_ref[...], preferred_element_type=jnp.float32)
    h2p_ref[...] = h2 * dis


def _tc2(acc, h1p, dis, w2, b1, g, be):
    return pl.pallas_call(
        _tc2_body,
        out_shape=jax.ShapeDtypeStruct((N, D), jnp.float32),
    )(acc, h1p, dis, w2, b1, g, be)


def _tc3_body(acc_ref, h2p_ref, dis_ref, b2_ref, out_ref):
    dis = dis_ref[...][:N]
    acc = acc_ref[0] + acc_ref[1]
    out_ref[...] = (acc[:N] + h2p_ref[...]) * dis + b2_ref[...]


def _tc3(acc, h2p, dis, b2):
    return pl.pallas_call(
        _tc3_body,
        out_shape=jax.ShapeDtypeStruct((N, D), jnp.float32),
    )(acc, h2p, dis, b2)


@jax.jit
def kernel(x, edge_index, W1, b1, gamma1, beta1, W2, b2):
    src = edge_index[0].reshape(NW, EPW)
    dst_w = edge_index[1].reshape(NW, EPW)
    dst = dst_w[:, :NCH * CH].reshape(NW, NCH, CH)
    dstt = dst_w[:, NCH * CH:]
    zeros_n = jnp.zeros((NPAD,), dtype=jnp.float32)
    zeros_rows = jnp.zeros((RPT, D), dtype=jnp.float32)

    degp = _deg_kernel(edge_index[1], zeros_n)
    h1p, dis = _tc1(x, W1, degp)
    acc1 = _agg_kernel(h1p, src, dst, dstt, zeros_rows)
    h2p = _tc2(acc1, h1p, dis, W2, b1.reshape(1, D), gamma1.reshape(1, D),
               beta1.reshape(1, D))
    acc2 = _agg_kernel(h2p, src, dst, dstt, zeros_rows)
    return _tc3(acc2, h2p, dis, b2.reshape(1, D))


# speedup vs baseline: 1.0749x; 1.0007x over previous
import functools

import jax
import jax.numpy as jnp
from jax import lax
from jax.experimental import pallas as pl
from jax.experimental.pallas import tpu as pltpu
from jax.experimental.pallas import tpu_sc as plsc

N = 10000
E = 320000
D = 128

NC = 2
NS = 16
NW = NC * NS
EPW = E // NW
CH = 104
NCH = 96
TAIL = EPW - NCH * CH
NPAD = 10112
RPT = NPAD // NS

_mesh = plsc.VectorSubcoreMesh(core_axis_name="c", subcore_axis_name="s")


@functools.partial(
    pl.kernel,
    out_type=jax.ShapeDtypeStruct((NW, NPAD), jnp.float32),
    mesh=_mesh,
    scratch_types=[
        pltpu.VMEM((EPW,), jnp.int32),
        pltpu.VMEM((NPAD,), jnp.float32),
        pltpu.SemaphoreType.DMA,
        pltpu.SemaphoreType.DMA,
    ],
    compiler_params=pltpu.CompilerParams(needs_layout_passes=False),
)
def _deg_kernel(dst_hbm, zeros_hbm, out_hbm, dst_v, deg_v, sem_a, sem_b):
    c = lax.axis_index("c")
    s = lax.axis_index("s")
    wid = s * NC + c
    p0 = pltpu.make_async_copy(dst_hbm.at[pl.ds(wid * EPW, EPW)], dst_v,
                               sem_a)
    p1 = pltpu.make_async_copy(zeros_hbm, deg_v, sem_b)
    p0.start()
    p1.start()
    p0.wait()
    p1.wait()
    ones = jnp.full((16,), 1.0, dtype=jnp.float32)

    def body(i, carry):
        base = i * 80
        for u in range(5):
            idx = dst_v[pl.ds(base + u * 16, 16)]
            plsc.addupdate_scatter(deg_v, [idx], ones)
        return carry

    lax.fori_loop(0, EPW // 80, body, 0)
    pltpu.sync_copy(deg_v, out_hbm.at[wid])


@functools.partial(
    pl.kernel,
    out_type=jax.ShapeDtypeStruct((NC, NPAD, D), jnp.float32),
    mesh=_mesh,
    scratch_types=[
        pltpu.VMEM((EPW,), jnp.int32),
        pltpu.VMEM((NCH, CH), jnp.int32),
        pltpu.VMEM((TAIL,), jnp.int32),
        pltpu.VMEM((CH, D), jnp.float32),
        pltpu.VMEM((CH, D), jnp.float32),
        pltpu.SemaphoreType.DMA,
        pltpu.SemaphoreType.DMA,
        pltpu.VMEM_SHARED((NPAD, D), jnp.float32),
    ],
)
def _agg_kernel(h_hbm, src_hbm, dst_hbm, dstt_hbm, zrows_hbm, out_hbm,
                src_v, dst_v, dstt_v, rows_a, rows_b, sem_a, sem_b, acc_sh):
    c = lax.axis_index("c")
    s = lax.axis_index("s")
    wid = s * NC + c
    p0 = pltpu.make_async_copy(zrows_hbm, acc_sh.at[pl.ds(s * RPT, RPT)],
                               sem_a)
    p1 = pltpu.make_async_copy(src_hbm.at[wid], src_v, sem_b)
    p2 = pltpu.make_async_copy(dst_hbm.at[wid], dst_v, sem_a)
    p3 = pltpu.make_async_copy(dstt_hbm.at[wid], dstt_v, sem_b)
    p0.start()
    p1.start()
    p2.start()
    p3.start()
    p0.wait()
    p1.wait()
    p2.wait()
    p3.wait()
    plsc.subcore_barrier()

    def g(j, rows, sem):
        return pltpu.make_async_copy(
            h_hbm.at[src_v.at[pl.ds(j * CH, CH)]], rows, sem)

    g(0, rows_a, sem_a).start()

    def step(k, carry):
        j0 = 2 * k
        g(j0 + 1, rows_b, sem_b).start()
        g(j0, rows_a, sem_a).wait()
        pltpu.sync_copy(rows_a, acc_sh.at[dst_v.at[j0]], add=True)
        g(j0 + 2, rows_a, sem_a).start()
        g(j0 + 1, rows_b, sem_b).wait()
        pltpu.sync_copy(rows_b, acc_sh.at[dst_v.at[j0 + 1]], add=True)
        return carry

    lax.fori_loop(0, (NCH - 2) // 2, step, 0)
    g(NCH - 1, rows_b, sem_b).start()
    g(NCH - 2, rows_a, sem_a).wait()
    pltpu.sync_copy(rows_a, acc_sh.at[dst_v.at[NCH - 2]], add=True)
    g(NCH - 1, rows_b, sem_b).wait()
    pltpu.sync_copy(rows_b, acc_sh.at[dst_v.at[NCH - 1]], add=True)
    tg = pltpu.make_async_copy(
        h_hbm.at[src_v.at[pl.ds(NCH * CH, TAIL)]],
        rows_a.at[pl.ds(0, TAIL)], sem_a)
    tg.start()
    tg.wait()
    pltpu.sync_copy(rows_a.at[pl.ds(0, TAIL)], acc_sh.at[dstt_v], add=True)
    plsc.subcore_barrier()
    pltpu.sync_copy(acc_sh.at[pl.ds(s * RPT, RPT)],
                    out_hbm.at[c, pl.ds(s * RPT, RPT)])


def _tc1_body(x_ref, w1_ref, degp_ref, h1p_ref, dis_ref):
    degp = degp_ref[...]
    ones = jnp.ones((NW, 1), dtype=jnp.float32)
    deg = lax.dot_general(degp, ones, (((0,), (0,)), ((), ())),
                          preferred_element_type=jnp.float32) + 1.0
    dis = lax.rsqrt(deg)
    h1 = jnp.dot(x_ref[...], w1_ref[...],
                 preferred_element_type=jnp.float32)
    h1p_ref[...] = h1 * dis[:N]
    dis_ref[...] = dis


def _tc1(x, w1, degp):
    return pl.pallas_call(
        _tc1_body,
        out_shape=(
            jax.ShapeDtypeStruct((N, D), jnp.float32),
            jax.ShapeDtypeStruct((NPAD, 1), jnp.float32),
        ),
    )(x, w1, degp)


def _tc2_body(acc_ref, h1p_ref, dis_ref, w2_ref, b1_ref, g_ref, be_ref,
              h2p_ref):
    dis = dis_ref[...][:N]
    acc = acc_ref[0] + acc_ref[1]
    out1 = (acc[:N] + h1p_ref[...]) * dis + b1_ref[...]
    mean = jnp.mean(out1, axis=0, keepdims=True)
    var = jnp.mean((out1 - mean) * (out1 - mean), axis=0, keepdims=True)
    hbn = (out1 - mean) * lax.rsqrt(var + 1e-5) * g_ref[...] + be_ref[...]
    hr = jnp.maximum(hbn, 0.0)
    h2 = jnp.dot(hr, w2_ref[...], preferred_element_type=jnp.float32)
    h2p_ref[...] = h2 * dis


def _tc2(acc, h1p, dis, w2, b1, g, be):
    return pl.pallas_call(
        _tc2_body,
        out_shape=jax.ShapeDtypeStruct((N, D), jnp.float32),
    )(acc, h1p, dis, w2, b1, g, be)


def _tc3_body(acc_ref, h2p_ref, dis_ref, b2_ref, out_ref):
    dis = dis_ref[...][:N]
    acc = acc_ref[0] + acc_ref[1]
    out_ref[...] = (acc[:N] + h2p_ref[...]) * dis + b2_ref[...]


def _tc3(acc, h2p, dis, b2):
    return pl.pallas_call(
        _tc3_body,
        out_shape=jax.ShapeDtypeStruct((N, D), jnp.float32),
    )(acc, h2p, dis, b2)


@jax.jit
def kernel(x, edge_index, W1, b1, gamma1, beta1, W2, b2):
    src = edge_index[0].reshape(NW, EPW)
    dst_w = edge_index[1].reshape(NW, EPW)
    dst = dst_w[:, :NCH * CH].reshape(NW, NCH, CH)
    dstt = dst_w[:, NCH * CH:]
    zeros_n = jnp.zeros((NPAD,), dtype=jnp.float32)
    zeros_rows = jnp.zeros((RPT, D), dtype=jnp.float32)

    degp = _deg_kernel(edge_index[1], zeros_n)
    h1p, dis = _tc1(x, W1, degp)
    acc1 = _agg_kernel(h1p, src, dst, dstt, zeros_rows)
    h2p = _tc2(acc1, h1p, dis, W2, b1.reshape(1, D), gamma1.reshape(1, D),
               beta1.reshape(1, D))
    acc2 = _agg_kernel(h2p, src, dst, dstt, zeros_rows)
    return _tc3(acc2, h2p, dis, b2.reshape(1, D))
